# Initial kernel scaffold; baseline (speedup 1.0000x reference)
import jax
import jax.numpy as jnp
from jax.experimental import pallas as pl


def kernel(image, patch_W, patch_b, cls_token, pos_embed, ln1_s, ln1_b, qkv_W, qkv_b, proj_W, proj_b, ln2_s, ln2_b, mlp_W1, mlp_b1, mlp_W2, mlp_b2, norm_s, norm_b, head_W, head_b):
    raise NotImplementedError("write your pallas kernel here")


# speedup vs baseline: 1.5242x; 1.5242x over previous
import functools

import jax
import jax.numpy as jnp
from jax.experimental import pallas as pl
from jax.experimental.pallas import tpu as pltpu

D = 768
H = 12
DH = 64
SCALE = 1.0 / (DH ** 0.5)
B = 8
NPAD_A = 200
NVALID_A = 197
NPAD_B = 104
NVALID_B = 99
NKEEP = 98
NC = 10
SQRT_HALF = 0.7071067811865476


def _ln(x, s, b):
    mu = jnp.mean(x, axis=-1, keepdims=True)
    var = jnp.mean((x - mu) ** 2, axis=-1, keepdims=True)
    return (x - mu) / jnp.sqrt(var + 1e-6) * s + b


def _attn_part(x, l1s, l1b, qw, qb, pw, pb, mask_bias):
    h = _ln(x, l1s, l1b)
    qkv = jnp.dot(h, qw, preferred_element_type=jnp.float32) + qb
    outs = []
    for hh in range(H):
        q = qkv[:, hh * DH:(hh + 1) * DH]
        k = qkv[:, D + hh * DH:D + (hh + 1) * DH]
        v = qkv[:, 2 * D + hh * DH:2 * D + (hh + 1) * DH]
        s = jax.lax.dot_general(q, k, (((1,), (1,)), ((), ())),
                                preferred_element_type=jnp.float32)
        s = s * SCALE + mask_bias
        m = jnp.max(s, axis=-1, keepdims=True)
        e = jnp.exp(s - m)
        p = e / jnp.sum(e, axis=-1, keepdims=True)
        outs.append(jnp.dot(p, v, preferred_element_type=jnp.float32))
    o = jnp.concatenate(outs, axis=1)
    return x + jnp.dot(o, pw, preferred_element_type=jnp.float32) + pb


def _mlp_half(xa, l2s, l2b, w1h, b1h, w2h):
    h2 = _ln(xa, l2s, l2b)
    g = jnp.dot(h2, w1h, preferred_element_type=jnp.float32) + b1h
    g = 0.5 * g * (1.0 + jax.lax.erf(g * SQRT_HALF))
    return jnp.dot(g, w2h, preferred_element_type=jnp.float32)


def _embed_body(xu_ref, w_ref, b_ref, cls_ref, pos_ref, out_ref):
    y = jnp.dot(xu_ref[...][0], w_ref[...],
                preferred_element_type=jnp.float32) + b_ref[...]
    y = y + pos_ref[...][1:, :]
    top = cls_ref[...] + pos_ref[...][0:1, :]
    pad = jnp.zeros((NPAD_A - NVALID_A, D), jnp.float32)
    out_ref[...] = jnp.concatenate([top, y, pad], axis=0)[None]


def _mask_bias(n_pad, n_valid):
    col = jax.lax.broadcasted_iota(jnp.int32, (1, n_pad), 1)
    return jnp.where(col >= n_valid, -1e30, 0.0).astype(jnp.float32)


def _prune_select(xnew):
    ones = jnp.ones((D, 1), jnp.float32)
    ncol = jax.lax.dot_general(jnp.abs(xnew), ones, (((1,), (0,)), ((), ())),
                               preferred_element_type=jnp.float32,
                               precision=jax.lax.Precision.HIGHEST)
    r = jax.lax.broadcasted_iota(jnp.int32, (NPAD_A, 1), 0)
    valid = (r >= 1) & (r < NVALID_A)
    ncol = jnp.where(valid, ncol, -1e30)
    nrow = ncol.reshape(1, NPAD_A)
    ji = jax.lax.broadcasted_iota(jnp.int32, (NPAD_A, NPAD_A), 0)
    ii = jax.lax.broadcasted_iota(jnp.int32, (NPAD_A, NPAD_A), 1)
    greater = (ncol > nrow) | ((ncol == nrow) & (ji < ii))
    rank = jnp.sum(greater.astype(jnp.float32), axis=0, keepdims=True)
    rank_i = rank.astype(jnp.int32)
    rr = jax.lax.broadcasted_iota(jnp.int32, (NPAD_B, NPAD_A), 0)
    cc = jax.lax.broadcasted_iota(jnp.int32, (NPAD_B, NPAD_A), 1)
    p1 = (rr == rank_i + 1) & (rr <= NKEEP)
    p0 = (rr == 0) & (cc == 0)
    return (p0 | p1).astype(jnp.float32)


def _layer_step(x_in, l1s, l1b, qw, qb, pw, pb, l2s, l2b, w1, b1, w2, b2,
                x_scr, xa_scr, n_pad, n_valid, tail_fn):
    l = pl.program_id(0)
    b = pl.program_id(1)
    k = pl.program_id(2)

    @pl.when(k == 0)
    def _():
        @pl.when(l == 0)
        def _():
            x_scr[pl.ds(b, 1)] = x_in[...]

        x = x_scr[pl.ds(b, 1)][0]
        mask = _mask_bias(n_pad, n_valid)
        xa = _attn_part(x, l1s[0], l1b[0], qw[0], qb[0], pw[0], pb[0], mask)
        xa_scr[...] = xa
        xmid = xa + b2[0] + _mlp_half(xa, l2s[0], l2b[0], w1[0], b1[0], w2[0])
        x_scr[pl.ds(b, 1)] = xmid[None]

    @pl.when(k == 1)
    def _():
        xa = xa_scr[...]
        xnew = x_scr[pl.ds(b, 1)][0] + _mlp_half(xa, l2s[0], l2b[0],
                                                 w1[0], b1[0], w2[0])
        x_scr[pl.ds(b, 1)] = xnew[None]

        @pl.when(l == pl.num_programs(0) - 1)
        def _():
            tail_fn(xnew)


def _phase_a_body(x_in, l1s, l1b, qw, qb, pw, pb, l2s, l2b, w1, b1, w2, b2,
                  out_ref, x_scr, xa_scr):
    def tail(xnew):
        P = _prune_select(xnew)
        sel = jnp.dot(P, xnew, preferred_element_type=jnp.float32,
                      precision=jax.lax.Precision.HIGHEST)
        out_ref[...] = sel[None, None]

    _layer_step(x_in, l1s, l1b, qw, qb, pw, pb, l2s, l2b, w1, b1,
                w2, b2, x_scr, xa_scr, NPAD_A, NVALID_A, tail)


def _phase_b_body(x_in, l1s, l1b, qw, qb, pw, pb, l2s, l2b, w1, b1, w2, b2,
                  ns, nb, hw, hb, out_ref, x_scr, xa_scr):
    def tail(xnew):
        cls = xnew[0:1, :]
        hcls = _ln(cls, ns[...], nb[...])
        lg = jnp.dot(hcls, hw[...], preferred_element_type=jnp.float32) + hb[...]
        out_ref[...] = lg[None, None]

    _layer_step(x_in, l1s, l1b, qw, qb, pw, pb, l2s, l2b, w1, b1,
                w2, b2, x_scr, xa_scr, NPAD_B, NVALID_B, tail)


def _wspec(shape, off):
    nd = len(shape)
    return pl.BlockSpec((1,) + shape[1:],
                        lambda l, b, k, _o=off, _n=nd: (l + _o,) + (0,) * (_n - 1))


def _layer_specs(off):
    return [
        _wspec((12, 1, D), off), _wspec((12, 1, D), off),
        _wspec((12, D, 3 * D), off), _wspec((12, 1, 3 * D), off),
        _wspec((12, D, D), off), _wspec((12, 1, D), off),
        _wspec((12, 1, D), off), _wspec((12, 1, D), off),
        pl.BlockSpec((1, D, 2 * D), lambda l, b, k, _o=off: (l + _o, 0, k)),
        pl.BlockSpec((1, 1, 2 * D), lambda l, b, k, _o=off: (l + _o, 0, k)),
        pl.BlockSpec((1, 2 * D, D), lambda l, b, k, _o=off: (l + _o, k, 0)),
        _wspec((12, 1, D), off),
    ]


def _xspec(n_pad):
    return pl.BlockSpec((1, n_pad, D), lambda l, b, k: (b, 0, 0))


def _const_spec(shape):
    nd = len(shape)
    return pl.BlockSpec(shape, lambda l, b, k, _n=nd: (0,) * _n)


def kernel(image, patch_W, patch_b, cls_token, pos_embed, ln1_s, ln1_b,
           qkv_W, qkv_b, proj_W, proj_b, ln2_s, ln2_b, mlp_W1, mlp_b1,
           mlp_W2, mlp_b2, norm_s, norm_b, head_W, head_b):
    G, P = 14, 16
    xu = image.reshape(B, 3, G, P, G, P).transpose(0, 2, 4, 1, 3, 5)
    xu = xu.reshape(B, G * G, 3 * P * P)

    x0 = pl.pallas_call(
        _embed_body,
        grid=(B,),
        in_specs=[
            pl.BlockSpec((1, G * G, 3 * P * P), lambda b: (b, 0, 0)),
            pl.BlockSpec((3 * P * P, D), lambda b: (0, 0)),
            pl.BlockSpec((1, D), lambda b: (0, 0)),
            pl.BlockSpec((1, D), lambda b: (0, 0)),
            pl.BlockSpec((NVALID_A, D), lambda b: (0, 0)),
        ],
        out_specs=pl.BlockSpec((1, NPAD_A, D), lambda b: (b, 0, 0)),
        out_shape=jax.ShapeDtypeStruct((B, NPAD_A, D), jnp.float32),
        compiler_params=pltpu.CompilerParams(
            dimension_semantics=("arbitrary",)),
    )(xu, patch_W, patch_b.reshape(1, D), cls_token.reshape(1, D),
      pos_embed.reshape(NVALID_A, D))

    lw = (ln1_s.reshape(12, 1, D), ln1_b.reshape(12, 1, D),
          qkv_W, qkv_b.reshape(12, 1, 3 * D),
          proj_W, proj_b.reshape(12, 1, D),
          ln2_s.reshape(12, 1, D), ln2_b.reshape(12, 1, D),
          mlp_W1, mlp_b1.reshape(12, 1, 4 * D),
          mlp_W2, mlp_b2.reshape(12, 1, D))

    xp = pl.pallas_call(
        _phase_a_body,
        grid=(7, B, 2),
        in_specs=[_xspec(NPAD_A)] + _layer_specs(0),
        out_specs=pl.BlockSpec((1, 1, NPAD_B, D), lambda l, b, k: (l, b, 0, 0)),
        out_shape=jax.ShapeDtypeStruct((7, B, NPAD_B, D), jnp.float32),
        scratch_shapes=[pltpu.VMEM((B, NPAD_A, D), jnp.float32),
                        pltpu.VMEM((NPAD_A, D), jnp.float32)],
        compiler_params=pltpu.CompilerParams(
            dimension_semantics=("arbitrary", "arbitrary", "arbitrary")),
    )(x0, *lw)
    xp = xp[6]

    logits = pl.pallas_call(
        _phase_b_body,
        grid=(5, B, 2),
        in_specs=([_xspec(NPAD_B)] + _layer_specs(7) +
                  [_const_spec((1, D)), _const_spec((1, D)),
                   _const_spec((D, NC)), _const_spec((1, NC))]),
        out_specs=pl.BlockSpec((1, 1, 1, NC), lambda l, b, k: (l, b, 0, 0)),
        out_shape=jax.ShapeDtypeStruct((5, B, 1, NC), jnp.float32),
        scratch_shapes=[pltpu.VMEM((B, NPAD_B, D), jnp.float32),
                        pltpu.VMEM((NPAD_B, D), jnp.float32)],
        compiler_params=pltpu.CompilerParams(
            dimension_semantics=("arbitrary", "arbitrary", "arbitrary")),
    )(xp, *lw, norm_s.reshape(1, D), norm_b.reshape(1, D),
      head_W, head_b.reshape(1, NC))

    return logits[4].reshape(B, NC)


# speedup vs baseline: 1.7281x; 1.1338x over previous
import functools

import jax
import jax.numpy as jnp
from jax.experimental import pallas as pl
from jax.experimental.pallas import tpu as pltpu
from jax.experimental.pallas import tpu_sc as plsc

D = 768
H = 12
DH = 64
SCALE = 1.0 / (DH ** 0.5)
B = 8
NPAD_A = 200
NVALID_A = 197
NPAD_B = 104
NVALID_B = 99
NKEEP = 98
NC = 10
SQRT_HALF = 0.7071067811865476


def _ln(x, s, b):
    mu = jnp.mean(x, axis=-1, keepdims=True)
    var = jnp.mean((x - mu) ** 2, axis=-1, keepdims=True)
    return (x - mu) / jnp.sqrt(var + 1e-6) * s + b


def _attn_part(x, l1s, l1b, qw, qb, pw, pb, mask_bias):
    h = _ln(x, l1s, l1b)
    qkv = jnp.dot(h, qw, preferred_element_type=jnp.float32) + qb
    outs = []
    for hh in range(H):
        q = qkv[:, hh * DH:(hh + 1) * DH]
        k = qkv[:, D + hh * DH:D + (hh + 1) * DH]
        v = qkv[:, 2 * D + hh * DH:2 * D + (hh + 1) * DH]
        s = jax.lax.dot_general(q, k, (((1,), (1,)), ((), ())),
                                preferred_element_type=jnp.float32)
        s = s * SCALE + mask_bias
        m = jnp.max(s, axis=-1, keepdims=True)
        e = jnp.exp(s - m)
        p = e / jnp.sum(e, axis=-1, keepdims=True)
        outs.append(jnp.dot(p, v, preferred_element_type=jnp.float32))
    o = jnp.concatenate(outs, axis=1)
    return x + jnp.dot(o, pw, preferred_element_type=jnp.float32) + pb


def _gelu_half(xa, l2s, l2b, w1h, b1h):
    h2 = _ln(xa, l2s, l2b)
    g = jnp.dot(h2, w1h, preferred_element_type=jnp.float32) + b1h
    return 0.5 * g * (1.0 + jax.lax.erf(g * SQRT_HALF))


def _embed_body(xu_ref, w_ref, b_ref, cls_ref, pos_ref, out_ref):
    y = jnp.dot(xu_ref[...][0], w_ref[...],
                preferred_element_type=jnp.float32) + b_ref[...]
    y = y + pos_ref[...][1:, :]
    top = cls_ref[...] + pos_ref[...][0:1, :]
    pad = jnp.zeros((NPAD_A - NVALID_A, D), jnp.float32)
    out_ref[...] = jnp.concatenate([top, y, pad], axis=0)[None]


def _mask_bias(n_pad, n_valid):
    col = jax.lax.broadcasted_iota(jnp.int32, (1, n_pad), 1)
    return jnp.where(col >= n_valid, -1e30, 0.0).astype(jnp.float32)


def _prune_select(xnew):
    ones = jnp.ones((D, 1), jnp.float32)
    ncol = jax.lax.dot_general(jnp.abs(xnew), ones, (((1,), (0,)), ((), ())),
                               preferred_element_type=jnp.float32,
                               precision=jax.lax.Precision.HIGHEST)
    r = jax.lax.broadcasted_iota(jnp.int32, (NPAD_A, 1), 0)
    valid = (r >= 1) & (r < NVALID_A)
    ncol = jnp.where(valid, ncol, -1e30)
    nrow = ncol.reshape(1, NPAD_A)
    ji = jax.lax.broadcasted_iota(jnp.int32, (NPAD_A, NPAD_A), 0)
    ii = jax.lax.broadcasted_iota(jnp.int32, (NPAD_A, NPAD_A), 1)
    greater = (ncol > nrow) | ((ncol == nrow) & (ji < ii))
    rank = jnp.sum(greater.astype(jnp.float32), axis=0, keepdims=True)
    rank_i = rank.astype(jnp.int32)
    rr = jax.lax.broadcasted_iota(jnp.int32, (NPAD_B, NPAD_A), 0)
    cc = jax.lax.broadcasted_iota(jnp.int32, (NPAD_B, NPAD_A), 1)
    p1 = (rr == rank_i + 1) & (rr <= NKEEP)
    p0 = (rr == 0) & (cc == 0)
    return (p0 | p1).astype(jnp.float32)


def _layer_step(x_in, l1s, l1b, qw, qb, pw, pb, l2s, l2b, w1, b1, w2, b2,
                x_scr, xa_scr, g_scr, n_pad, n_valid, tail_fn):
    l = pl.program_id(0)
    b = pl.program_id(1)
    k = pl.program_id(2)

    @pl.when(k == 0)
    def _():
        @pl.when(l == 0)
        def _():
            x_scr[pl.ds(b, 1)] = x_in[...]

        x = x_scr[pl.ds(b, 1)][0]
        mask = _mask_bias(n_pad, n_valid)
        xa = _attn_part(x, l1s[0], l1b[0], qw[0], qb[0], pw[0], pb[0], mask)
        xa_scr[...] = xa
        g_scr[:, 0:2 * D] = _gelu_half(xa, l2s[0], l2b[0], w1[0], b1[0])

    @pl.when(k == 1)
    def _():
        xa = xa_scr[...]
        g_scr[:, 2 * D:4 * D] = _gelu_half(xa, l2s[0], l2b[0], w1[0], b1[0])
        g = g_scr[...]
        xnew = xa + b2[0] + jnp.dot(g, w2[0],
                                    preferred_element_type=jnp.float32)
        x_scr[pl.ds(b, 1)] = xnew[None]

        @pl.when(l == pl.num_programs(0) - 1)
        def _():
            tail_fn(xnew)


NIDX = 128


def _phase_a_body(x_in, l1s, l1b, qw, qb, pw, pb, l2s, l2b, w1, b1, w2, b2,
                  x_out, idx_out, x_scr, xa_scr, g_scr):
    def tail(xnew):
        P = _prune_select(xnew)
        rowids = (jax.lax.broadcasted_iota(jnp.int32, (NPAD_A, 1), 0)
                  + pl.program_id(1) * NPAD_A).astype(jnp.float32)
        gidx = jnp.dot(P, rowids, preferred_element_type=jnp.float32,
                       precision=jax.lax.Precision.HIGHEST)
        gi = jnp.concatenate(
            [gidx.reshape(1, NPAD_B),
             jnp.zeros((1, NIDX - NPAD_B), jnp.float32)], axis=1)
        x_out[...] = xnew[None, None]
        idx_out[...] = gi.astype(jnp.int32)[None, None]

    _layer_step(x_in, l1s, l1b, qw, qb, pw, pb, l2s, l2b, w1, b1,
                w2, b2, x_scr, xa_scr, g_scr, NPAD_A, NVALID_A, tail)


_TOT = B * NIDX


def _sc_gather(x_flat, idx_flat):
    info = plsc.get_sparse_core_info()
    ncores = info.num_cores
    nw = ncores * info.num_subcores
    bpw = _TOT // nw
    mesh = plsc.VectorSubcoreMesh(core_axis_name="c", subcore_axis_name="s")

    @functools.partial(
        pl.kernel, mesh=mesh,
        out_type=jax.ShapeDtypeStruct((_TOT, D), jnp.float32),
        scratch_types=[pltpu.VMEM((bpw,), jnp.int32),
                       pltpu.VMEM((bpw, D), jnp.float32),
                       pltpu.SemaphoreType.DMA])
    def gat(x_hbm, idx_hbm, out_hbm, idx_v, rows_v, sem):
        wid = jax.lax.axis_index("s") * ncores + jax.lax.axis_index("c")
        base = wid * bpw
        pltpu.sync_copy(idx_hbm.at[pl.ds(base, bpw)], idx_v)
        pltpu.async_copy(x_hbm.at[idx_v], rows_v, sem).wait()
        pltpu.sync_copy(rows_v, out_hbm.at[pl.ds(base, bpw)])

    return gat(x_flat, idx_flat)


def _phase_b_body(x_in, l1s, l1b, qw, qb, pw, pb, l2s, l2b, w1, b1, w2, b2,
                  ns, nb, hw, hb, out_ref, x_scr, xa_scr, g_scr):
    def tail(xnew):
        cls = xnew[0:1, :]
        hcls = _ln(cls, ns[...], nb[...])
        lg = jnp.dot(hcls, hw[...], preferred_element_type=jnp.float32) + hb[...]
        out_ref[...] = lg[None, None]

    _layer_step(x_in, l1s, l1b, qw, qb, pw, pb, l2s, l2b, w1, b1,
                w2, b2, x_scr, xa_scr, g_scr, NPAD_B, NVALID_B, tail)


def _wspec(shape, off):
    nd = len(shape)
    return pl.BlockSpec((1,) + shape[1:],
                        lambda l, b, k, _o=off, _n=nd: (l + _o,) + (0,) * (_n - 1))


def _layer_specs(off):
    return [
        _wspec((12, 1, D), off), _wspec((12, 1, D), off),
        _wspec((12, D, 3 * D), off), _wspec((12, 1, 3 * D), off),
        _wspec((12, D, D), off), _wspec((12, 1, D), off),
        _wspec((12, 1, D), off), _wspec((12, 1, D), off),
        pl.BlockSpec((1, D, 2 * D), lambda l, b, k, _o=off: (l + _o, 0, k)),
        pl.BlockSpec((1, 1, 2 * D), lambda l, b, k, _o=off: (l + _o, 0, k)),
        _wspec((12, 4 * D, D), off),
        _wspec((12, 1, D), off),
    ]


def _xspec(n_pad):
    return pl.BlockSpec((1, n_pad, D), lambda l, b, k: (b, 0, 0))


def _const_spec(shape):
    nd = len(shape)
    return pl.BlockSpec(shape, lambda l, b, k, _n=nd: (0,) * _n)


def kernel(image, patch_W, patch_b, cls_token, pos_embed, ln1_s, ln1_b,
           qkv_W, qkv_b, proj_W, proj_b, ln2_s, ln2_b, mlp_W1, mlp_b1,
           mlp_W2, mlp_b2, norm_s, norm_b, head_W, head_b):
    G, P = 14, 16
    xu = image.reshape(B, 3, G, P, G, P).transpose(0, 2, 4, 1, 3, 5)
    xu = xu.reshape(B, G * G, 3 * P * P)

    x0 = pl.pallas_call(
        _embed_body,
        grid=(B,),
        in_specs=[
            pl.BlockSpec((1, G * G, 3 * P * P), lambda b: (b, 0, 0)),
            pl.BlockSpec((3 * P * P, D), lambda b: (0, 0)),
            pl.BlockSpec((1, D), lambda b: (0, 0)),
            pl.BlockSpec((1, D), lambda b: (0, 0)),
            pl.BlockSpec((NVALID_A, D), lambda b: (0, 0)),
        ],
        out_specs=pl.BlockSpec((1, NPAD_A, D), lambda b: (b, 0, 0)),
        out_shape=jax.ShapeDtypeStruct((B, NPAD_A, D), jnp.float32),
        compiler_params=pltpu.CompilerParams(
            dimension_semantics=("arbitrary",)),
    )(xu, patch_W, patch_b.reshape(1, D), cls_token.reshape(1, D),
      pos_embed.reshape(NVALID_A, D))

    lw = (ln1_s.reshape(12, 1, D), ln1_b.reshape(12, 1, D),
          qkv_W, qkv_b.reshape(12, 1, 3 * D),
          proj_W, proj_b.reshape(12, 1, D),
          ln2_s.reshape(12, 1, D), ln2_b.reshape(12, 1, D),
          mlp_W1, mlp_b1.reshape(12, 1, 4 * D),
          mlp_W2, mlp_b2.reshape(12, 1, D))

    xa_full, gidx = pl.pallas_call(
        _phase_a_body,
        grid=(7, B, 2),
        in_specs=[_xspec(NPAD_A)] + _layer_specs(0),
        out_specs=[
            pl.BlockSpec((1, 1, NPAD_A, D), lambda l, b, k: (l, b, 0, 0)),
            pl.BlockSpec((1, 1, 1, NIDX), lambda l, b, k: (l, b, 0, 0)),
        ],
        out_shape=[
            jax.ShapeDtypeStruct((7, B, NPAD_A, D), jnp.float32),
            jax.ShapeDtypeStruct((7, B, 1, NIDX), jnp.int32),
        ],
        scratch_shapes=[pltpu.VMEM((B, NPAD_A, D), jnp.float32),
                        pltpu.VMEM((NPAD_A, D), jnp.float32),
                        pltpu.VMEM((NPAD_A, 4 * D), jnp.float32)],
        compiler_params=pltpu.CompilerParams(
            dimension_semantics=("arbitrary", "arbitrary", "arbitrary")),
    )(x0, *lw)

    xg = _sc_gather(xa_full[6].reshape(B * NPAD_A, D),
                    gidx[6].reshape(_TOT))
    xp = xg.reshape(B, NIDX, D)[:, :NPAD_B]

    logits = pl.pallas_call(
        _phase_b_body,
        grid=(5, B, 2),
        in_specs=([_xspec(NPAD_B)] + _layer_specs(7) +
                  [_const_spec((1, D)), _const_spec((1, D)),
                   _const_spec((D, NC)), _const_spec((1, NC))]),
        out_specs=pl.BlockSpec((1, 1, 1, NC), lambda l, b, k: (l, b, 0, 0)),
        out_shape=jax.ShapeDtypeStruct((5, B, 1, NC), jnp.float32),
        scratch_shapes=[pltpu.VMEM((B, NPAD_B, D), jnp.float32),
                        pltpu.VMEM((NPAD_B, D), jnp.float32),
                        pltpu.VMEM((NPAD_B, 4 * D), jnp.float32)],
        compiler_params=pltpu.CompilerParams(
            dimension_semantics=("arbitrary", "arbitrary", "arbitrary")),
    )(xp, *lw, norm_s.reshape(1, D), norm_b.reshape(1, D),
      head_W, head_b.reshape(1, NC))

    return logits[4].reshape(B, NC)
